# Initial kernel scaffold; baseline (speedup 1.0000x reference)
import jax
import jax.numpy as jnp
from jax.experimental import pallas as pl


def kernel(inputs, rel_embeddings):
    raise NotImplementedError("write your pallas kernel here")


# speedup vs baseline: 6.7882x; 6.7882x over previous
import functools

import jax
import jax.numpy as jnp
from jax import lax
from jax.experimental import pallas as pl
from jax.experimental.pallas import tpu as pltpu
from jax.experimental.pallas import tpu_sc as plsc

_MAX_POSITION = 2048


def _sc_expand(rev_hbm, out_hbm, tab, sem_in, sem_out, *, length, rows_per, fire_k):
    c = lax.axis_index("c")
    s = lax.axis_index("s")

    @pl.when(s == 0)
    def _():
        pltpu.async_copy(rev_hbm, tab, sem_in).wait()

    plsc.subcore_barrier()

    wid = c * 16 + s
    base = wid * rows_per
    for chunk in range(rows_per // fire_k):
        handles = []
        for r in range(fire_k):
            i = base + chunk * fire_k + r
            start = (length - 1) - i
            handles.append(
                pltpu.async_copy(tab.at[pl.ds(start, length)], out_hbm.at[i], sem_out)
            )
        for h in handles:
            h.wait()


def kernel(inputs, rel_embeddings):
    length = inputs.shape[1]
    depth = rel_embeddings.shape[1]
    table_rows = rel_embeddings.shape[0]

    rev = rel_embeddings[::-1]

    n_workers = 32
    rows_per = length // n_workers

    mesh = plsc.VectorSubcoreMesh(core_axis_name="c", subcore_axis_name="s")
    body = functools.partial(
        _sc_expand, length=length, rows_per=rows_per, fire_k=8
    )
    out = pl.kernel(
        body,
        mesh=mesh,
        out_type=jax.ShapeDtypeStruct((length, length, depth), jnp.float32),
        scratch_types=[
            pltpu.VMEM_SHARED((table_rows, depth), jnp.float32),
            pltpu.SemaphoreType.DMA,
            pltpu.SemaphoreType.DMA,
        ],
    )(rev)
    return out


# speedup vs baseline: 6.7933x; 1.0008x over previous
import functools

import jax
import jax.numpy as jnp
from jax import lax
from jax.experimental import pallas as pl
from jax.experimental.pallas import tpu as pltpu
from jax.experimental.pallas import tpu_sc as plsc

_MAX_POSITION = 2048


def _sc_expand(rev_hbm, out_hbm, tab, sem_in, sem_out, *, length, rows_per, fire_k):
    c = lax.axis_index("c")
    s = lax.axis_index("s")

    @pl.when(s == 0)
    def _():
        pltpu.async_copy(rev_hbm, tab, sem_in).wait()

    plsc.subcore_barrier()

    wid = c * 16 + s
    base = wid * rows_per
    for chunk in range(rows_per // fire_k):
        handles = []
        for r in range(fire_k):
            i = base + chunk * fire_k + r
            start = (length - 1) - i
            handles.append(
                pltpu.async_copy(tab.at[pl.ds(start, length)], out_hbm.at[i], sem_out)
            )
        for h in handles:
            h.wait()


def kernel(inputs, rel_embeddings):
    length = inputs.shape[1]
    depth = rel_embeddings.shape[1]
    table_rows = rel_embeddings.shape[0]

    rev = rel_embeddings[::-1]

    n_workers = 32
    rows_per = length // n_workers

    mesh = plsc.VectorSubcoreMesh(core_axis_name="c", subcore_axis_name="s")
    body = functools.partial(
        _sc_expand, length=length, rows_per=rows_per, fire_k=8
    )
    out = pl.kernel(
        body,
        mesh=mesh,
        compiler_params=pltpu.CompilerParams(use_tc_tiling_on_sc=True),
        out_type=jax.ShapeDtypeStruct((length, length, depth), jnp.float32),
        scratch_types=[
            pltpu.VMEM_SHARED((table_rows, depth), jnp.float32),
            pltpu.SemaphoreType.DMA,
            pltpu.SemaphoreType.DMA,
        ],
    )(rev)
    return out
